# baseline (speedup 1.0000x reference)
import functools

import jax
import jax.numpy as jnp
from jax import lax
from jax.experimental import pallas as pl
from jax.experimental.pallas import tpu as pltpu
from jax.experimental.pallas import tpu_sc as plsc

_D = 4096
_N = 64
_LO = (_D - _N) // 2
_BLK = 512
_NBLK = _D // _BLK
_NW = 32
_CHUNK = _D // _NW
_LANES = 16


def _mlp_metric_kernel(xw_ref, w1w_ref, b1_ref, b2_ref, wm_ref, w2_ref,
                       t2_ref, m_ref, h_ref):
    j = pl.program_id(0)

    @pl.when(j == 0)
    def _init():
        pre = jnp.dot(xw_ref[...], w1w_ref[...],
                      preferred_element_type=jnp.float32)
        h_ref[...] = jnp.maximum(pre + b1_ref[...] + w1w_ref[...], 0.0)
        m_ref[...] = jnp.zeros_like(m_ref)

    t2_blk = jnp.dot(h_ref[...], w2_ref[...],
                     preferred_element_type=jnp.float32) + b2_ref[...]
    t2_ref[...] = t2_blk
    m_ref[...] += jnp.sum(t2_blk * wm_ref[...], axis=1, keepdims=True)


@functools.partial(
    pl.kernel,
    mesh=plsc.VectorSubcoreMesh(core_axis_name="c", subcore_axis_name="s"),
    out_type=jax.ShapeDtypeStruct((_D,), jnp.float32),
    scratch_types=[
        pltpu.VMEM((_N,), jnp.float32),
        pltpu.VMEM((_CHUNK,), jnp.float32),
    ],
)
def _sc_argmax_gather(metric_hbm, t2_hbm, out_hbm, met_v, row_v):
    wid = lax.axis_index("c") * 16 + lax.axis_index("s")
    pltpu.sync_copy(metric_hbm, met_v)
    vals = []
    for c in range(_N // _LANES):
        v = met_v[pl.ds(c * _LANES, _LANES)]
        vals.extend(v[l] for l in range(_LANES))
    m = vals[0]
    idx = jnp.int32(0)
    for i in range(1, _N):
        better = vals[i] > m
        m = jnp.where(better, vals[i], m)
        idx = jnp.where(better, jnp.int32(i), idx)
    col = wid * _CHUNK
    pltpu.sync_copy(t2_hbm.at[idx, pl.ds(col, _CHUNK)], row_v)
    pltpu.sync_copy(row_v, out_hbm.at[pl.ds(col, _CHUNK)])


@jax.jit
def kernel(x, W1, b1, W2, b2, w_metric):
    xw = jax.lax.slice(x, (_LO,), (_LO + _N,)).reshape(1, _N)
    w1w = jax.lax.slice(W1, (_LO, 0), (_LO + _N, _D))
    b1r = b1.reshape(1, _D)
    b2r = b2.reshape(1, _D)
    wmr = w_metric.reshape(1, _D)

    t2, metric = pl.pallas_call(
        _mlp_metric_kernel,
        grid=(_NBLK,),
        in_specs=[
            pl.BlockSpec((1, _N), lambda j: (0, 0)),
            pl.BlockSpec((_N, _D), lambda j: (0, 0)),
            pl.BlockSpec((1, _D), lambda j: (0, 0)),
            pl.BlockSpec((1, _BLK), lambda j: (0, j)),
            pl.BlockSpec((1, _BLK), lambda j: (0, j)),
            pl.BlockSpec((_D, _BLK), lambda j: (0, j)),
        ],
        out_specs=[
            pl.BlockSpec((_N, _BLK), lambda j: (0, j)),
            pl.BlockSpec((_N, 1), lambda j: (0, 0)),
        ],
        out_shape=[
            jax.ShapeDtypeStruct((_N, _D), jnp.float32),
            jax.ShapeDtypeStruct((_N, 1), jnp.float32),
        ],
        scratch_shapes=[
            pltpu.VMEM((_N, _D), jnp.float32),
        ],
        compiler_params=pltpu.CompilerParams(
            dimension_semantics=("arbitrary",),
        ),
    )(xw, w1w, b1r, b2r, wmr, W2)

    return _sc_argmax_gather(metric.reshape(_N), t2)


# speedup vs baseline: 1.0534x; 1.0534x over previous
import functools

import jax
import jax.numpy as jnp
from jax import lax
from jax.experimental import pallas as pl
from jax.experimental.pallas import tpu as pltpu
from jax.experimental.pallas import tpu_sc as plsc

_D = 4096
_N = 64
_LO = (_D - _N) // 2
_BLK = 1024
_NBLK = _D // _BLK
_NW = 32
_CHUNK = _D // _NW
_LANES = 16


def _mlp_metric_kernel(x_ref, w1a_ref, w1b_ref, b1_ref, b2_ref, wm_ref,
                       w2_ref, t2_ref, m_ref, h_ref):
    j = pl.program_id(0)

    @pl.when(j == 0)
    def _init():
        w1w = jnp.concatenate([w1a_ref[...], w1b_ref[...]], axis=0)
        xw = x_ref[:, _LO:_LO + _N]
        pre = jnp.dot(xw, w1w, preferred_element_type=jnp.float32)
        h_ref[...] = jnp.maximum(pre + b1_ref[...] + w1w, 0.0)
        m_ref[...] = jnp.zeros_like(m_ref)

    t2_blk = jnp.dot(h_ref[...], w2_ref[...],
                     preferred_element_type=jnp.float32) + b2_ref[...]
    t2_ref[...] = t2_blk
    m_ref[...] += jnp.sum(t2_blk * wm_ref[...], axis=1, keepdims=True)


@functools.partial(
    pl.kernel,
    mesh=plsc.VectorSubcoreMesh(core_axis_name="c", subcore_axis_name="s"),
    out_type=jax.ShapeDtypeStruct((_D,), jnp.float32),
    scratch_types=[
        pltpu.VMEM((_N,), jnp.float32),
        pltpu.VMEM((_CHUNK,), jnp.float32),
    ],
)
def _sc_argmax_gather(metric_hbm, t2_hbm, out_hbm, met_v, row_v):
    wid = lax.axis_index("c") * 16 + lax.axis_index("s")
    pltpu.sync_copy(metric_hbm, met_v)
    vals = []
    for c in range(_N // _LANES):
        v = met_v[pl.ds(c * _LANES, _LANES)]
        vals.extend(v[l] for l in range(_LANES))
    m = vals[0]
    idx = jnp.int32(0)
    for i in range(1, _N):
        better = vals[i] > m
        m = jnp.where(better, vals[i], m)
        idx = jnp.where(better, jnp.int32(i), idx)
    col = wid * _CHUNK
    pltpu.sync_copy(t2_hbm.at[idx, pl.ds(col, _CHUNK)], row_v)
    pltpu.sync_copy(row_v, out_hbm.at[pl.ds(col, _CHUNK)])


@jax.jit
def kernel(x, W1, b1, W2, b2, w_metric):
    xr = x.reshape(1, _D)
    b1r = b1.reshape(1, _D)
    b2r = b2.reshape(1, _D)
    wmr = w_metric.reshape(1, _D)

    t2, metric = pl.pallas_call(
        _mlp_metric_kernel,
        grid=(_NBLK,),
        in_specs=[
            pl.BlockSpec((1, _D), lambda j: (0, 0)),
            pl.BlockSpec((32, _D), lambda j: (_LO // 32, 0)),
            pl.BlockSpec((32, _D), lambda j: (_LO // 32 + 1, 0)),
            pl.BlockSpec((1, _D), lambda j: (0, 0)),
            pl.BlockSpec((1, _BLK), lambda j: (0, j)),
            pl.BlockSpec((1, _BLK), lambda j: (0, j)),
            pl.BlockSpec((_D, _BLK), lambda j: (0, j)),
        ],
        out_specs=[
            pl.BlockSpec((_N, _BLK), lambda j: (0, j)),
            pl.BlockSpec((_N, 1), lambda j: (0, 0)),
        ],
        out_shape=[
            jax.ShapeDtypeStruct((_N, _D), jnp.float32),
            jax.ShapeDtypeStruct((_N, 1), jnp.float32),
        ],
        scratch_shapes=[
            pltpu.VMEM((_N, _D), jnp.float32),
        ],
        compiler_params=pltpu.CompilerParams(
            dimension_semantics=("arbitrary",),
        ),
    )(xr, W1, W1, b1r, b2r, wmr, W2)

    return _sc_argmax_gather(metric.reshape(_N), t2)


# speedup vs baseline: 1.9630x; 1.8634x over previous
import jax
import jax.numpy as jnp
from jax.experimental import pallas as pl
from jax.experimental.pallas import tpu as pltpu

_D = 4096
_N = 64
_LO = (_D - _N) // 2
_BLK = 1024
_NBLK = _D // _BLK


def _mlp_argmax_kernel(x_ref, w1a_ref, w1b_ref, b1_ref, b2_ref, wm_ref,
                       w2_ref, out_ref, h_ref, t2_ref, m_ref):
    j = pl.program_id(0)

    @pl.when(j == 0)
    def _init():
        w1w = jnp.concatenate([w1a_ref[...], w1b_ref[...]], axis=0)
        xw = x_ref[:, _LO:_LO + _N]
        pre = jnp.dot(xw, w1w, preferred_element_type=jnp.float32)
        h_ref[...] = jnp.maximum(pre + b1_ref[...] + w1w, 0.0)
        m_ref[...] = jnp.zeros_like(m_ref)

    t2_blk = jnp.dot(h_ref[...], w2_ref[...],
                     preferred_element_type=jnp.float32) + b2_ref[...]
    t2_ref[:, pl.ds(j * _BLK, _BLK)] = t2_blk
    m_ref[...] += jnp.sum(t2_blk * wm_ref[...], axis=1, keepdims=True)

    @pl.when(j == _NBLK - 1)
    def _fin():
        metric = m_ref[...]
        mmax = jnp.max(metric)
        iota = jax.lax.broadcasted_iota(jnp.int32, (_N, 1), 0)
        idx = jnp.min(jnp.where(metric == mmax, iota, _N))
        onehot = (iota == idx).astype(jnp.float32)
        out_ref[...] = jnp.sum(t2_ref[...] * onehot, axis=0, keepdims=True)


@jax.jit
def kernel(x, W1, b1, W2, b2, w_metric):
    xr = x.reshape(1, _D)
    b1r = b1.reshape(1, _D)
    b2r = b2.reshape(1, _D)
    wmr = w_metric.reshape(1, _D)

    out = pl.pallas_call(
        _mlp_argmax_kernel,
        grid=(_NBLK,),
        in_specs=[
            pl.BlockSpec((1, _D), lambda j: (0, 0)),
            pl.BlockSpec((32, _D), lambda j: (_LO // 32, 0)),
            pl.BlockSpec((32, _D), lambda j: (_LO // 32 + 1, 0)),
            pl.BlockSpec((1, _D), lambda j: (0, 0)),
            pl.BlockSpec((1, _BLK), lambda j: (0, j)),
            pl.BlockSpec((1, _BLK), lambda j: (0, j)),
            pl.BlockSpec((_D, _BLK), lambda j: (0, j)),
        ],
        out_specs=pl.BlockSpec((1, _D), lambda j: (0, 0)),
        out_shape=jax.ShapeDtypeStruct((1, _D), jnp.float32),
        scratch_shapes=[
            pltpu.VMEM((_N, _D), jnp.float32),
            pltpu.VMEM((_N, _D), jnp.float32),
            pltpu.VMEM((_N, 1), jnp.float32),
        ],
        compiler_params=pltpu.CompilerParams(
            dimension_semantics=("arbitrary",),
        ),
    )(xr, W1, W1, b1r, b2r, wmr, W2)
    return out.reshape(_D)


# speedup vs baseline: 1.9747x; 1.0060x over previous
import jax
import jax.numpy as jnp
from jax.experimental import pallas as pl
from jax.experimental.pallas import tpu as pltpu

_D = 4096
_N = 64
_LO = (_D - _N) // 2
_BLK = 1024
_NBLK = _D // _BLK


def _mlp_argmax_kernel(x_ref, w1a_ref, w1b_ref, b1_ref, b2_ref, wm_ref,
                       w2_ref, out_ref, h_ref, t2_ref):
    j = pl.program_id(0)

    @pl.when(j == 0)
    def _init():
        w1w = jnp.concatenate([w1a_ref[...], w1b_ref[...]], axis=0)
        xw = x_ref[:, _LO:_LO + _N]
        pre = jnp.dot(xw, w1w, preferred_element_type=jnp.float32)
        h_ref[...] = jnp.maximum(pre + b1_ref[...] + w1w, 0.0)
        t2_ref[...] = jnp.broadcast_to(b2_ref[...], (_N, _D))

    h_blk = h_ref[:, pl.ds(j * _BLK, _BLK)]
    t2_ref[...] += jnp.dot(h_blk, w2_ref[...],
                           preferred_element_type=jnp.float32)

    @pl.when(j == _NBLK - 1)
    def _fin():
        t2 = t2_ref[...]
        metric = jnp.sum(t2 * wm_ref[...], axis=1, keepdims=True)
        mmax = jnp.max(metric)
        iota = jax.lax.broadcasted_iota(jnp.int32, (_N, 1), 0)
        idx = jnp.min(jnp.where(metric == mmax, iota, _N))
        onehot = (iota == idx).astype(jnp.float32)
        out_ref[...] = jnp.sum(t2 * onehot, axis=0, keepdims=True)


@jax.jit
def kernel(x, W1, b1, W2, b2, w_metric):
    xr = x.reshape(1, _D)
    b1r = b1.reshape(1, _D)
    b2r = b2.reshape(1, _D)
    wmr = w_metric.reshape(1, _D)

    out = pl.pallas_call(
        _mlp_argmax_kernel,
        grid=(_NBLK,),
        in_specs=[
            pl.BlockSpec((1, _D), lambda j: (0, 0)),
            pl.BlockSpec((32, _D), lambda j: (_LO // 32, 0)),
            pl.BlockSpec((32, _D), lambda j: (_LO // 32 + 1, 0)),
            pl.BlockSpec((1, _D), lambda j: (0, 0)),
            pl.BlockSpec((1, _D), lambda j: (0, 0)),
            pl.BlockSpec((1, _D), lambda j: (0, 0)),
            pl.BlockSpec((_BLK, _D), lambda j: (j, 0)),
        ],
        out_specs=pl.BlockSpec((1, _D), lambda j: (0, 0)),
        out_shape=jax.ShapeDtypeStruct((1, _D), jnp.float32),
        scratch_shapes=[
            pltpu.VMEM((_N, _D), jnp.float32),
            pltpu.VMEM((_N, _D), jnp.float32),
        ],
        compiler_params=pltpu.CompilerParams(
            dimension_semantics=("arbitrary",),
        ),
    )(xr, W1, W1, b1r, b2r, wmr, W2)
    return out.reshape(_D)


# speedup vs baseline: 1.9766x; 1.0009x over previous
import jax
import jax.numpy as jnp
from jax.experimental import pallas as pl
from jax.experimental.pallas import tpu as pltpu

_D = 4096
_N = 64
_LO = (_D - _N) // 2
_BLK = 1024
_NBLK = _D // _BLK


def _mlp_argmax_kernel(x_ref, w1a_ref, w1b_ref, b1_ref, b2_ref, wm_ref,
                       w2_ref, out_ref, h_ref, t2_ref, m_ref):
    j = pl.program_id(0)

    @pl.when(j == 0)
    def _init():
        w1w = jnp.concatenate([w1a_ref[...], w1b_ref[...]], axis=0)
        xw = x_ref[:, _LO:_LO + _N]
        pre = jnp.dot(xw, w1w, preferred_element_type=jnp.float32)
        h_ref[...] = jnp.maximum(pre + b1_ref[...] + w1w, 0.0)
        b2 = jnp.broadcast_to(b2_ref[...], (_N, _D))
        t2_ref[...] = b2
        m_ref[...] = jnp.sum(b2 * wm_ref[...], axis=1, keepdims=True)

    h_blk = h_ref[:, pl.ds(j * _BLK, _BLK)]
    delta = jnp.dot(h_blk, w2_ref[...], preferred_element_type=jnp.float32)
    t2_ref[...] += delta
    m_ref[...] += jnp.sum(delta * wm_ref[...], axis=1, keepdims=True)

    @pl.when(j == _NBLK - 1)
    def _fin():
        metric = m_ref[...]
        mmax = jnp.max(metric)
        iota = jax.lax.broadcasted_iota(jnp.int32, (_N, 1), 0)
        idx = jnp.min(jnp.where(metric == mmax, iota, _N))
        out_ref[...] = t2_ref[pl.ds(idx, 1), :]


@jax.jit
def kernel(x, W1, b1, W2, b2, w_metric):
    xr = x.reshape(1, _D)
    b1r = b1.reshape(1, _D)
    b2r = b2.reshape(1, _D)
    wmr = w_metric.reshape(1, _D)

    out = pl.pallas_call(
        _mlp_argmax_kernel,
        grid=(_NBLK,),
        in_specs=[
            pl.BlockSpec((1, _D), lambda j: (0, 0)),
            pl.BlockSpec((32, _D), lambda j: (_LO // 32, 0)),
            pl.BlockSpec((32, _D), lambda j: (_LO // 32 + 1, 0)),
            pl.BlockSpec((1, _D), lambda j: (0, 0)),
            pl.BlockSpec((1, _D), lambda j: (0, 0)),
            pl.BlockSpec((1, _D), lambda j: (0, 0)),
            pl.BlockSpec((_BLK, _D), lambda j: (j, 0)),
        ],
        out_specs=pl.BlockSpec((1, _D), lambda j: (0, 0)),
        out_shape=jax.ShapeDtypeStruct((1, _D), jnp.float32),
        scratch_shapes=[
            pltpu.VMEM((_N, _D), jnp.float32),
            pltpu.VMEM((_N, _D), jnp.float32),
            pltpu.VMEM((_N, 1), jnp.float32),
        ],
        compiler_params=pltpu.CompilerParams(
            dimension_semantics=("arbitrary",),
        ),
    )(xr, W1, W1, b1r, b2r, wmr, W2)
    return out.reshape(_D)


# speedup vs baseline: 2.1017x; 1.0633x over previous
import jax
import jax.numpy as jnp
from jax.experimental import pallas as pl
from jax.experimental.pallas import tpu as pltpu

_D = 4096
_N = 64
_LO = (_D - _N) // 2
_BLK = 512
_NBLK = _D // _BLK


def _mlp_argmax_kernel(x_ref, w1a_ref, w1b_ref, b1_ref, b2_ref, wm_ref,
                       w2_ref, out_ref, h_ref, t2_ref, m_ref):
    j = pl.program_id(0)

    @pl.when(j == 0)
    def _init():
        w1w = jnp.concatenate([w1a_ref[...], w1b_ref[...]], axis=0)
        xw = x_ref[:, _LO:_LO + _N]
        pre = jnp.dot(xw, w1w, preferred_element_type=jnp.float32)
        h_ref[...] = jnp.maximum(pre + b1_ref[...] + w1w, 0.0)
        b2 = jnp.broadcast_to(b2_ref[...], (_N, _D))
        t2_ref[...] = b2
        m_ref[...] = jnp.sum(b2 * wm_ref[...], axis=1, keepdims=True)

    h_blk = h_ref[:, pl.ds(j * _BLK, _BLK)]
    delta = jnp.dot(h_blk, w2_ref[...], preferred_element_type=jnp.float32)
    t2_ref[...] += delta
    m_ref[...] += jnp.sum(delta * wm_ref[...], axis=1, keepdims=True)

    @pl.when(j == _NBLK - 1)
    def _fin():
        metric = m_ref[...]
        mmax = jnp.max(metric)
        iota = jax.lax.broadcasted_iota(jnp.int32, (_N, 1), 0)
        idx = jnp.min(jnp.where(metric == mmax, iota, _N))
        out_ref[...] = t2_ref[pl.ds(idx, 1), :]


@jax.jit
def kernel(x, W1, b1, W2, b2, w_metric):
    xr = x.reshape(1, _D)
    b1r = b1.reshape(1, _D)
    b2r = b2.reshape(1, _D)
    wmr = w_metric.reshape(1, _D)

    out = pl.pallas_call(
        _mlp_argmax_kernel,
        grid=(_NBLK,),
        in_specs=[
            pl.BlockSpec((1, _D), lambda j: (0, 0)),
            pl.BlockSpec((32, _D), lambda j: (_LO // 32, 0)),
            pl.BlockSpec((32, _D), lambda j: (_LO // 32 + 1, 0)),
            pl.BlockSpec((1, _D), lambda j: (0, 0)),
            pl.BlockSpec((1, _D), lambda j: (0, 0)),
            pl.BlockSpec((1, _D), lambda j: (0, 0)),
            pl.BlockSpec((_BLK, _D), lambda j: (j, 0)),
        ],
        out_specs=pl.BlockSpec((1, _D), lambda j: (0, 0)),
        out_shape=jax.ShapeDtypeStruct((1, _D), jnp.float32),
        scratch_shapes=[
            pltpu.VMEM((_N, _D), jnp.float32),
            pltpu.VMEM((_N, _D), jnp.float32),
            pltpu.VMEM((_N, 1), jnp.float32),
        ],
        compiler_params=pltpu.CompilerParams(
            dimension_semantics=("arbitrary",),
        ),
    )(xr, W1, W1, b1r, b2r, wmr, W2)
    return out.reshape(_D)
